# Initial kernel scaffold; baseline (speedup 1.0000x reference)
import jax
import jax.numpy as jnp
from jax.experimental import pallas as pl


def kernel(x, edge_index, edge_attr, W1, b1, W2, b2, Wl, bl):
    raise NotImplementedError("write your pallas kernel here")


# speedup vs baseline: 178.8256x; 178.8256x over previous
import functools

import jax
import jax.numpy as jnp
from jax import lax
from jax.experimental import pallas as pl
from jax.experimental.pallas import tpu as pltpu
from jax.experimental.pallas import tpu_sc as plsc

N = 10000
NP = 10240
E = 320000
F_IN = 128
LANES = 16
CHUNK = 2000

_MESH = plsc.VectorSubcoreMesh(core_axis_name="c", subcore_axis_name="s")
_SC_PARAMS = pltpu.CompilerParams(needs_layout_passes=False)
_NTILES = 32

_R1, _R2, _R3 = 2, 4, 8
_R4, _R5, _R6 = 10, 16, 32


def _wid():
    return lax.axis_index("s") * 2 + lax.axis_index("c")


def _fori(n, body):
    lax.fori_loop(jnp.int32(0), jnp.int32(n), body, jnp.int32(0))


def _zero_col(col):
    z = jnp.zeros((LANES,), jnp.float32)

    def body(i, _):
        col[pl.ds(i * LANES, LANES)] = z
        return jnp.int32(0)

    _fori(NP // LANES, body)


def _load_reduced_col(tin, col_idx, rp, in_col, tmp_col):
    pltpu.sync_copy(tin.at[jnp.int32(0), col_idx], in_col)
    for p in range(1, rp):
        pltpu.sync_copy(tin.at[jnp.int32(p), col_idx], tmp_col)

        def body(i, _):
            s = pl.ds(i * LANES, LANES)
            in_col[s] = in_col[s] + tmp_col[s]
            return jnp.int32(0)

        _fori(NP // LANES, body)


def _make_prop(d, off, rp, dprev, nranges):
    er = E // nranges
    items = d * nranges
    assert items <= _NTILES and er % CHUNK == 0

    @functools.partial(
        pl.kernel,
        out_type=jax.ShapeDtypeStruct((nranges, d, NP), jnp.float32),
        mesh=_MESH,
        compiler_params=_SC_PARAMS,
        scratch_types=[
            pltpu.VMEM((NP,), jnp.float32),
            pltpu.VMEM((NP,), jnp.float32),
            pltpu.VMEM((NP,), jnp.float32),
            pltpu.VMEM((CHUNK,), jnp.int32),
            pltpu.VMEM((CHUNK,), jnp.int32),
            pltpu.VMEM((CHUNK,), jnp.float32),
        ],
    )
    def prop(tin, src, dst, nrm, out, in_col, tmp_col, acc_col, src_v, dst_v,
             nrm_v):
        wid = _wid()

        @pl.when(wid < items)
        def _():
            cidx = wid % d
            ridx = wid // d
            _load_reduced_col(tin, off + cidx, rp, in_col, tmp_col)
            _zero_col(acc_col)
            ebase = ridx * er

            def chunk_body(ci, _):
                o = ebase + ci * CHUNK
                pltpu.sync_copy(src.at[pl.ds(o, CHUNK)], src_v)
                pltpu.sync_copy(dst.at[pl.ds(o, CHUNK)], dst_v)
                pltpu.sync_copy(nrm.at[pl.ds(o, CHUNK)], nrm_v)

                def vbody(j, _):
                    s = pl.ds(j * LANES, LANES)
                    g = plsc.load_gather(in_col, [src_v[s]])
                    plsc.addupdate_scatter(acc_col, [dst_v[s]], g * nrm_v[s])
                    return jnp.int32(0)

                _fori(CHUNK // LANES, vbody)
                return jnp.int32(0)

            _fori(er // CHUNK, chunk_body)
            pltpu.sync_copy(acc_col, out.at[ridx, cidx])

    return prop


@functools.partial(
    pl.kernel,
    out_type=jax.ShapeDtypeStruct((_NTILES, NP), jnp.float32),
    mesh=_MESH,
    compiler_params=_SC_PARAMS,
    scratch_types=[
        pltpu.VMEM((NP,), jnp.float32),
        pltpu.VMEM((CHUNK,), jnp.int32),
        pltpu.VMEM((CHUNK,), jnp.float32),
    ],
)
def _deg_kernel(dst, ew, out, acc_col, dst_v, ew_v):
    wid = _wid()
    er = E // _NTILES
    _zero_col(acc_col)
    ebase = wid * er

    def chunk_body(ci, _):
        o = ebase + ci * CHUNK
        pltpu.sync_copy(dst.at[pl.ds(o, CHUNK)], dst_v)
        pltpu.sync_copy(ew.at[pl.ds(o, CHUNK)], ew_v)

        def vbody(j, _):
            s = pl.ds(j * LANES, LANES)
            plsc.addupdate_scatter(acc_col, [dst_v[s]], ew_v[s])
            return jnp.int32(0)

        _fori(CHUNK // LANES, vbody)
        return jnp.int32(0)

    _fori(er // CHUNK, chunk_body)
    pltpu.sync_copy(acc_col, out.at[wid])


@functools.partial(
    pl.kernel,
    out_type=jax.ShapeDtypeStruct((E,), jnp.float32),
    mesh=_MESH,
    compiler_params=_SC_PARAMS,
    scratch_types=[
        pltpu.VMEM((NP,), jnp.float32),
        pltpu.VMEM((CHUNK,), jnp.int32),
        pltpu.VMEM((CHUNK,), jnp.int32),
        pltpu.VMEM((CHUNK,), jnp.float32),
        pltpu.VMEM((CHUNK,), jnp.float32),
    ],
)
def _norm_kernel(dis, src, dst, ew, out, dis_col, src_v, dst_v, ew_v, nrm_v):
    wid = _wid()
    er = E // _NTILES
    pltpu.sync_copy(dis, dis_col)
    ebase = wid * er

    def chunk_body(ci, _):
        o = ebase + ci * CHUNK
        pltpu.sync_copy(src.at[pl.ds(o, CHUNK)], src_v)
        pltpu.sync_copy(dst.at[pl.ds(o, CHUNK)], dst_v)
        pltpu.sync_copy(ew.at[pl.ds(o, CHUNK)], ew_v)

        def vbody(j, _):
            s = pl.ds(j * LANES, LANES)
            g1 = plsc.load_gather(dis_col, [src_v[s]])
            g2 = plsc.load_gather(dis_col, [dst_v[s]])
            nrm_v[s] = g1 * ew_v[s] * g2
            return jnp.int32(0)

        _fori(CHUNK // LANES, vbody)
        pltpu.sync_copy(nrm_v, out.at[pl.ds(o, CHUNK)])
        return jnp.int32(0)

    _fori(er // CHUNK, chunk_body)


def _proj1_body(x_ref, w_ref, out_ref):
    out_ref[...] = lax.dot_general(
        w_ref[...], x_ref[...], (((0,), (1,)), ((), ())),
        preferred_element_type=jnp.float32)


def _dis_body(degp_ref, out_ref):
    deg = jnp.sum(degp_ref[...], axis=0, keepdims=True)
    out_ref[...] = jnp.where(deg > 0.0, lax.rsqrt(deg), 0.0)


def _mid_body(y0_ref, p1_ref, p2_ref, p3_ref, b1_ref, w2_ref, out_ref):
    h = (y0_ref[...] + jnp.sum(p1_ref[...], axis=0)
         + jnp.sum(p2_ref[...], axis=0) + jnp.sum(p3_ref[...], axis=0)
         + b1_ref[...])
    h = jnp.maximum(h, 0.0)
    out_ref[...] = lax.dot_general(
        w2_ref[...], h, (((1,), (0,)), ((), ())),
        preferred_element_type=jnp.float32)


def _fin_body(v0_ref, q1_ref, q2_ref, q3_ref, b2_ref, wl_ref, bl_ref,
              out_ref):
    o = (v0_ref[...] + jnp.sum(q1_ref[...], axis=0, keepdims=True)
         + jnp.sum(q2_ref[...], axis=0, keepdims=True)
         + jnp.sum(q3_ref[...], axis=0, keepdims=True) + b2_ref[...])
    o = o * wl_ref[...] + bl_ref[...]
    out_ref[...] = 1.0 / (1.0 + jnp.exp(-o))


def kernel(x, edge_index, edge_attr, W1, b1, W2, b2, Wl, bl):
    src = edge_index[0].astype(jnp.int32)
    dst = edge_index[1].astype(jnp.int32)
    ew = edge_attr.astype(jnp.float32)
    xpad = jnp.zeros((NP, F_IN), jnp.float32).at[:N].set(
        x.astype(jnp.float32))
    wcat1 = jnp.concatenate([W1[0], W1[1], W1[2], W1[3]],
                            axis=1).astype(jnp.float32)

    yt = pl.pallas_call(
        _proj1_body,
        out_shape=jax.ShapeDtypeStruct((16, NP), jnp.float32),
    )(xpad, wcat1)

    degp = _deg_kernel(dst, ew)
    dis = pl.pallas_call(
        _dis_body,
        out_shape=jax.ShapeDtypeStruct((1, NP), jnp.float32),
    )(degp)

    nrm = _norm_kernel(dis.reshape(NP), src, dst, ew)

    p1 = _make_prop(12, 4, 1, 16, _R1)(yt.reshape(1, 16, NP), src, dst, nrm)
    p2 = _make_prop(8, 4, _R1, 12, _R2)(p1, src, dst, nrm)
    p3 = _make_prop(4, 4, _R2, 8, _R3)(p2, src, dst, nrm)

    w2mat = W2[:, :, 0].astype(jnp.float32)
    v = pl.pallas_call(
        _mid_body,
        out_shape=jax.ShapeDtypeStruct((4, NP), jnp.float32),
    )(yt[0:4], p1[:, 0:4], p2[:, 0:4], p3, b1.reshape(4, 1), w2mat)

    q1 = _make_prop(3, 1, 1, 4, _R4)(v.reshape(1, 4, NP), src, dst, nrm)
    q2 = _make_prop(2, 1, _R4, 3, _R5)(q1, src, dst, nrm)
    q3 = _make_prop(1, 1, _R5, 2, _R6)(q2, src, dst, nrm)

    o = pl.pallas_call(
        _fin_body,
        out_shape=jax.ShapeDtypeStruct((1, NP), jnp.float32),
    )(v[0:1], q1[:, 0], q2[:, 0], q3[:, 0],
      b2.reshape(1, 1).astype(jnp.float32), Wl.astype(jnp.float32),
      bl.reshape(1, 1).astype(jnp.float32))

    return o[0, :N].reshape(N, 1).astype(jnp.float64)


# speedup vs baseline: 487.5380x; 2.7263x over previous
import functools

import jax
import jax.numpy as jnp
from jax import lax
from jax.experimental import pallas as pl
from jax.experimental.pallas import tpu as pltpu
from jax.experimental.pallas import tpu_sc as plsc

N = 10000
NP = 10240
E = 320000
F_IN = 128
LANES = 16
UNROLL = 10

_MESH = plsc.VectorSubcoreMesh(core_axis_name="c", subcore_axis_name="s")
_SC_PARAMS = pltpu.CompilerParams(needs_layout_passes=False)
_NTILES = 32


def _wid():
    return lax.axis_index("s") * 2 + lax.axis_index("c")


def _fori(n, body):
    lax.fori_loop(jnp.int32(0), jnp.int32(n), body, jnp.int32(0))


def _zero_col(col):
    z = jnp.zeros((LANES,), jnp.float32)

    def body(i, _):
        base = i * (8 * LANES)
        for u in range(8):
            col[pl.ds(base + u * LANES, LANES)] = z
        return jnp.int32(0)

    _fori(NP // (8 * LANES), body)


def _add_into(dst_col, src_col):
    def body(i, _):
        base = i * (8 * LANES)
        for u in range(8):
            s = pl.ds(base + u * LANES, LANES)
            dst_col[s] = dst_col[s] + src_col[s]
        return jnp.int32(0)

    _fori(NP // (8 * LANES), body)


def _load_col_reduced(base, bcol, part, rp, in_col, tmps, sems, pcol):
    pltpu.async_copy(base.at[bcol], in_col, sems[0])
    pltpu.make_async_copy(base.at[bcol], in_col, sems[0]).wait()
    for i, p in enumerate(range(rp)):
        if i == 0:
            pltpu.async_copy(part.at[jnp.int32(p), pcol], tmps[0], sems[0])
        if i + 1 < rp:
            s2 = (i + 1) % 2
            pltpu.async_copy(part.at[jnp.int32(p + 1), pcol], tmps[s2],
                             sems[s2])
        sl = i % 2
        pltpu.make_async_copy(part.at[jnp.int32(p), pcol], tmps[sl],
                              sems[sl]).wait()
        _add_into(in_col, tmps[sl])


def _make_prop(d, boff, rp, base_d, nranges, chunk):
    er = E // nranges
    items = d * nranges
    nchunks = er // chunk
    gpc = chunk // LANES
    unroll = UNROLL if gpc % UNROLL == 0 else 5
    assert er % chunk == 0 and gpc % unroll == 0
    assert items <= _NTILES

    @functools.partial(
        pl.kernel,
        out_type=jax.ShapeDtypeStruct((nranges, d, NP), jnp.float32),
        mesh=_MESH,
        compiler_params=_SC_PARAMS,
        scratch_types=(
            [pltpu.VMEM((NP,), jnp.float32) for _ in range(4)]
            + [pltpu.VMEM((chunk,), jnp.int32),
               pltpu.VMEM((chunk,), jnp.int32),
               pltpu.VMEM((chunk,), jnp.int32),
               pltpu.VMEM((chunk,), jnp.int32),
               pltpu.VMEM((chunk,), jnp.float32),
               pltpu.VMEM((chunk,), jnp.float32),
               pltpu.SemaphoreType.DMA,
               pltpu.SemaphoreType.DMA]
        ),
    )
    def prop(*args):
        if rp:
            (base, part, src, dst, nrm, out, in_col, acc_col, tmp0, tmp1,
             src_v0, src_v1, dst_v0, dst_v1, nrm_v0, nrm_v1,
             sem0, sem1) = args
        else:
            (base, src, dst, nrm, out, in_col, acc_col, tmp0, tmp1,
             src_v0, src_v1, dst_v0, dst_v1, nrm_v0, nrm_v1,
             sem0, sem1) = args
            part = None
        tmps = (tmp0, tmp1)
        svs, dvs, nvs = (src_v0, src_v1), (dst_v0, dst_v1), (nrm_v0, nrm_v1)
        sems = (sem0, sem1)
        wid = _wid()

        @pl.when(wid < items)
        def _():
            cidx = wid % d
            ridx = wid // d
            _load_col_reduced(base, jnp.int32(boff) + cidx, part, rp,
                              in_col, tmps, sems, cidx)
            _zero_col(acc_col)
            ebase = ridx * er

            def issue(ci, slot):
                o = ebase + ci * chunk
                pltpu.async_copy(src.at[pl.ds(o, chunk)], svs[slot],
                                 sems[slot])
                pltpu.async_copy(dst.at[pl.ds(o, chunk)], dvs[slot],
                                 sems[slot])
                pltpu.async_copy(nrm.at[pl.ds(o, chunk)], nvs[slot],
                                 sems[slot])

            def drain(ci, slot):
                o = ebase + ci * chunk
                pltpu.make_async_copy(src.at[pl.ds(o, chunk)],
                                      svs[slot], sems[slot]).wait()
                pltpu.make_async_copy(dst.at[pl.ds(o, chunk)],
                                      dvs[slot], sems[slot]).wait()
                pltpu.make_async_copy(nrm.at[pl.ds(o, chunk)],
                                      nvs[slot], sems[slot]).wait()

            issue(0, 0)
            if nchunks > 1:
                issue(1, 1)
            for ci in range(nchunks):
                slot = ci % 2
                drain(ci, slot)
                sv, dv, nv = svs[slot], dvs[slot], nvs[slot]

                def vbody(j, _):
                    jb = j * (unroll * LANES)
                    for u in range(unroll):
                        s = pl.ds(jb + u * LANES, LANES)
                        si = sv[s]
                        di = dv[s]
                        w = nv[s]
                        gth = plsc.load_gather(in_col, [si])
                        plsc.addupdate_scatter(acc_col, [di], gth * w)
                    return jnp.int32(0)

                _fori(gpc // unroll, vbody)
                if ci + 2 < nchunks:
                    issue(ci + 2, slot)

            pltpu.sync_copy(acc_col, out.at[ridx, cidx])

    return prop


_DEG_TILES = 20
_DEG_ER = E // _DEG_TILES
_DEG_CHUNK = 4000
_DEG_NC = _DEG_ER // _DEG_CHUNK


@functools.partial(
    pl.kernel,
    out_type=jax.ShapeDtypeStruct((_DEG_TILES, NP), jnp.float32),
    mesh=_MESH,
    compiler_params=_SC_PARAMS,
    scratch_types=[
        pltpu.VMEM((NP,), jnp.float32),
        pltpu.VMEM((_DEG_CHUNK,), jnp.int32),
        pltpu.VMEM((_DEG_CHUNK,), jnp.int32),
        pltpu.VMEM((_DEG_CHUNK,), jnp.float32),
        pltpu.VMEM((_DEG_CHUNK,), jnp.float32),
        pltpu.SemaphoreType.DMA,
        pltpu.SemaphoreType.DMA,
    ],
)
def _deg_kernel(dst, ew, out, acc_col, dst_v0, dst_v1, ew_v0, ew_v1,
                sem0, sem1):
    wid = _wid()
    dvs, wvs = (dst_v0, dst_v1), (ew_v0, ew_v1)
    sems = (sem0, sem1)

    @pl.when(wid < _DEG_TILES)
    def _():
        _zero_col(acc_col)
        ebase = wid * _DEG_ER

        def issue(ci, slot):
            o = ebase + ci * _DEG_CHUNK
            pltpu.async_copy(dst.at[pl.ds(o, _DEG_CHUNK)], dvs[slot],
                             sems[slot])
            pltpu.async_copy(ew.at[pl.ds(o, _DEG_CHUNK)], wvs[slot],
                             sems[slot])

        def drain(ci, slot):
            o = ebase + ci * _DEG_CHUNK
            pltpu.make_async_copy(dst.at[pl.ds(o, _DEG_CHUNK)],
                                  dvs[slot], sems[slot]).wait()
            pltpu.make_async_copy(ew.at[pl.ds(o, _DEG_CHUNK)],
                                  wvs[slot], sems[slot]).wait()

        issue(0, 0)
        issue(1, 1)
        for ci in range(_DEG_NC):
            slot = ci % 2
            drain(ci, slot)
            dv, wv = dvs[slot], wvs[slot]

            def vbody(j, _):
                base = j * (UNROLL * LANES)
                for u in range(UNROLL):
                    s = pl.ds(base + u * LANES, LANES)
                    plsc.addupdate_scatter(acc_col, [dv[s]], wv[s])
                return jnp.int32(0)

            _fori(_DEG_CHUNK // LANES // UNROLL, vbody)
            if ci + 2 < _DEG_NC:
                issue(ci + 2, slot)

        pltpu.sync_copy(acc_col, out.at[wid])


@functools.partial(
    pl.kernel,
    out_type=jax.ShapeDtypeStruct((E,), jnp.float32),
    mesh=_MESH,
    compiler_params=_SC_PARAMS,
    scratch_types=[
        pltpu.VMEM((NP,), jnp.float32),
        pltpu.VMEM((_DEG_CHUNK,), jnp.int32),
        pltpu.VMEM((_DEG_CHUNK,), jnp.int32),
        pltpu.VMEM((_DEG_CHUNK,), jnp.int32),
        pltpu.VMEM((_DEG_CHUNK,), jnp.int32),
        pltpu.VMEM((_DEG_CHUNK,), jnp.float32),
        pltpu.VMEM((_DEG_CHUNK,), jnp.float32),
        pltpu.VMEM((_DEG_CHUNK,), jnp.float32),
        pltpu.VMEM((_DEG_CHUNK,), jnp.float32),
        pltpu.SemaphoreType.DMA,
        pltpu.SemaphoreType.DMA,
        pltpu.SemaphoreType.DMA,
    ],
)
def _norm_kernel(dis, src, dst, ew, out, dis_col, src_v0, src_v1,
                 dst_v0, dst_v1, ew_v0, ew_v1, nrm_v0, nrm_v1,
                 sem0, sem1, sem_st):
    wid = _wid()
    svs, dvs = (src_v0, src_v1), (dst_v0, dst_v1)
    wvs, nvs = (ew_v0, ew_v1), (nrm_v0, nrm_v1)
    sems = (sem0, sem1)

    @pl.when(wid < _DEG_TILES)
    def _():
        pltpu.sync_copy(dis, dis_col)
        ebase = wid * _DEG_ER

        def issue(ci, slot):
            o = ebase + ci * _DEG_CHUNK
            pltpu.async_copy(src.at[pl.ds(o, _DEG_CHUNK)], svs[slot],
                             sems[slot])
            pltpu.async_copy(dst.at[pl.ds(o, _DEG_CHUNK)], dvs[slot],
                             sems[slot])
            pltpu.async_copy(ew.at[pl.ds(o, _DEG_CHUNK)], wvs[slot],
                             sems[slot])

        def drain(ci, slot):
            o = ebase + ci * _DEG_CHUNK
            pltpu.make_async_copy(src.at[pl.ds(o, _DEG_CHUNK)],
                                  svs[slot], sems[slot]).wait()
            pltpu.make_async_copy(dst.at[pl.ds(o, _DEG_CHUNK)],
                                  dvs[slot], sems[slot]).wait()
            pltpu.make_async_copy(ew.at[pl.ds(o, _DEG_CHUNK)],
                                  wvs[slot], sems[slot]).wait()

        def wait_store(ci, slot):
            o = ebase + ci * _DEG_CHUNK
            pltpu.make_async_copy(nvs[slot],
                                  out.at[pl.ds(o, _DEG_CHUNK)],
                                  sem_st).wait()

        issue(0, 0)
        issue(1, 1)
        for ci in range(_DEG_NC):
            slot = ci % 2
            drain(ci, slot)
            if ci >= 2:
                wait_store(ci - 2, slot)
            sv, dv, wv, nv = svs[slot], dvs[slot], wvs[slot], nvs[slot]

            def vbody(j, _):
                base = j * (UNROLL * LANES)
                for u in range(UNROLL):
                    s = pl.ds(base + u * LANES, LANES)
                    g1 = plsc.load_gather(dis_col, [sv[s]])
                    g2 = plsc.load_gather(dis_col, [dv[s]])
                    nv[s] = g1 * wv[s] * g2
                return jnp.int32(0)

            _fori(_DEG_CHUNK // LANES // UNROLL, vbody)
            o = ebase + ci * _DEG_CHUNK
            pltpu.async_copy(nvs[slot], out.at[pl.ds(o, _DEG_CHUNK)],
                             sem_st)
            if ci + 2 < _DEG_NC:
                issue(ci + 2, slot)

        for ci in range(max(_DEG_NC - 2, 0), _DEG_NC):
            wait_store(ci, ci % 2)


def _proj1_body(x_ref, w_ref, out_ref):
    out_ref[...] = lax.dot_general(
        w_ref[...], x_ref[...], (((0,), (1,)), ((), ())),
        preferred_element_type=jnp.float32)


def _dis_body(degp_ref, out_ref):
    deg = jnp.sum(degp_ref[...], axis=0, keepdims=True)
    out_ref[...] = jnp.where(deg > 0.0, lax.rsqrt(deg), 0.0)


def _mid_body(y0_ref, t3_ref, b1_ref, w2_ref, out_ref):
    h = y0_ref[...] + jnp.sum(t3_ref[...], axis=0) + b1_ref[...]
    h = jnp.maximum(h, 0.0)
    out_ref[...] = lax.dot_general(
        w2_ref[...], h, (((1,), (0,)), ((), ())),
        preferred_element_type=jnp.float32)


def _fin_body(v0_ref, q_ref, b2_ref, wl_ref, bl_ref, out_ref):
    o = (v0_ref[...] + jnp.sum(q_ref[...], axis=0, keepdims=True)
         + b2_ref[...])
    o = o * wl_ref[...] + bl_ref[...]
    out_ref[...] = 1.0 / (1.0 + jnp.exp(-o))


_P1 = (4, 12, 0, 16, 8, 4000)
_P2 = (4, 8, 8, 16, 8, 4000)
_P3 = (4, 4, 8, 16, 8, 4000)
_P4 = (1, 3, 0, 4, 16, 2000)
_P5 = (1, 2, 16, 4, 16, 2000)
_P6 = (1, 1, 16, 4, 16, 2000)


def kernel(x, edge_index, edge_attr, W1, b1, W2, b2, Wl, bl):
    src = edge_index[0].astype(jnp.int32)
    dst = edge_index[1].astype(jnp.int32)
    ew = edge_attr.astype(jnp.float32)
    xpad = jnp.zeros((NP, F_IN), jnp.float32).at[:N].set(
        x.astype(jnp.float32))
    wcat1 = jnp.concatenate([W1[0], W1[1], W1[2], W1[3]],
                            axis=1).astype(jnp.float32)

    yt = pl.pallas_call(
        _proj1_body,
        out_shape=jax.ShapeDtypeStruct((16, NP), jnp.float32),
    )(xpad, wcat1)

    degp = _deg_kernel(dst, ew)
    dis = pl.pallas_call(
        _dis_body,
        out_shape=jax.ShapeDtypeStruct((1, NP), jnp.float32),
    )(degp)

    nrm = _norm_kernel(dis.reshape(NP), src, dst, ew)

    t = _make_prop(*_P1)(yt, src, dst, nrm)
    t = _make_prop(*_P2)(yt, t, src, dst, nrm)
    t = _make_prop(*_P3)(yt, t, src, dst, nrm)

    w2mat = W2[:, :, 0].astype(jnp.float32)
    v = pl.pallas_call(
        _mid_body,
        out_shape=jax.ShapeDtypeStruct((4, NP), jnp.float32),
    )(yt[0:4], t, b1.reshape(4, 1), w2mat)

    q = _make_prop(*_P4)(v, src, dst, nrm)
    q = _make_prop(*_P5)(v, q, src, dst, nrm)
    q = _make_prop(*_P6)(v, q, src, dst, nrm)

    o = pl.pallas_call(
        _fin_body,
        out_shape=jax.ShapeDtypeStruct((1, NP), jnp.float32),
    )(v[0:1], q[:, 0],
      b2.reshape(1, 1).astype(jnp.float32), Wl.astype(jnp.float32),
      bl.reshape(1, 1).astype(jnp.float32))

    return o[0, :N].reshape(N, 1).astype(jnp.float64)


# speedup vs baseline: 552.6760x; 1.1336x over previous
import functools

import jax
import jax.numpy as jnp
from jax import lax
from jax.experimental import pallas as pl
from jax.experimental.pallas import tpu as pltpu
from jax.experimental.pallas import tpu_sc as plsc

N = 10000
NP = 10240
E = 320000
F_IN = 128
LANES = 16
UNROLL = 10

_MESH = plsc.VectorSubcoreMesh(core_axis_name="c", subcore_axis_name="s")
_SC_PARAMS = pltpu.CompilerParams(needs_layout_passes=False)
_NTILES = 32


def _wid():
    return lax.axis_index("s") * 2 + lax.axis_index("c")


def _fori(n, body):
    lax.fori_loop(jnp.int32(0), jnp.int32(n), body, jnp.int32(0))


def _zero_col(col):
    z = jnp.zeros((LANES,), jnp.float32)

    def body(i, _):
        base = i * (8 * LANES)
        for u in range(8):
            col[pl.ds(base + u * LANES, LANES)] = z
        return jnp.int32(0)

    _fori(NP // (8 * LANES), body)


def _add_into(dst_col, src_col):
    def body(i, _):
        base = i * (8 * LANES)
        for u in range(8):
            s = pl.ds(base + u * LANES, LANES)
            dst_col[s] = dst_col[s] + src_col[s]
        return jnp.int32(0)

    _fori(NP // (8 * LANES), body)


def _load_col_reduced(base, bcol, part, plist, in_col, tmps, sems):
    pltpu.async_copy(base.at[bcol], in_col, sems[0])
    pltpu.make_async_copy(base.at[bcol], in_col, sems[0]).wait()
    for i, (pi, pc) in enumerate(plist):
        if i == 0:
            pltpu.async_copy(part.at[pi, pc], tmps[0], sems[0])
        if i + 1 < len(plist):
            pi2, pc2 = plist[i + 1]
            s2 = (i + 1) % 2
            pltpu.async_copy(part.at[pi2, pc2], tmps[s2], sems[s2])
        sl = i % 2
        pltpu.make_async_copy(part.at[pi, pc], tmps[sl], sems[sl]).wait()
        _add_into(in_col, tmps[sl])


def _make_prop(d, boff, rp, base_d, nranges, chunk):
    assert d in (1, 4)
    ncpc = max(d // 2, 1)
    nranges = 8 if d == 4 else 32
    er = E // nranges
    nchunks = er // chunk
    gpc = chunk // LANES
    unroll = UNROLL if gpc % UNROLL == 0 else 5
    nper = 16 // ncpc
    segw = NP // nper
    assert er % chunk == 0 and gpc % unroll == 0 and NP % nper == 0

    @functools.partial(
        pl.kernel,
        out_type=jax.ShapeDtypeStruct((2, d, NP), jnp.float32),
        mesh=_MESH,
        compiler_params=_SC_PARAMS,
        scratch_types=(
            [pltpu.VMEM((NP,), jnp.float32) for _ in range(4)]
            + [pltpu.VMEM_SHARED((16, NP), jnp.float32),
               pltpu.VMEM((segw,), jnp.float32),
               pltpu.VMEM((segw,), jnp.float32),
               pltpu.VMEM((chunk,), jnp.int32),
               pltpu.VMEM((chunk,), jnp.int32),
               pltpu.VMEM((chunk,), jnp.int32),
               pltpu.VMEM((chunk,), jnp.int32),
               pltpu.VMEM((chunk,), jnp.float32),
               pltpu.VMEM((chunk,), jnp.float32),
               pltpu.SemaphoreType.DMA,
               pltpu.SemaphoreType.DMA]
        ),
    )
    def prop(*args):
        if rp:
            (base, part, src, dst, nrm, out, in_col, acc_col, tmp0, tmp1,
             shared, seg_a, seg_b,
             src_v0, src_v1, dst_v0, dst_v1, nrm_v0, nrm_v1,
             sem0, sem1) = args
        else:
            (base, src, dst, nrm, out, in_col, acc_col, tmp0, tmp1,
             shared, seg_a, seg_b,
             src_v0, src_v1, dst_v0, dst_v1, nrm_v0, nrm_v1,
             sem0, sem1) = args
            part = None
        tmps = (tmp0, tmp1)
        svs, dvs, nvs = (src_v0, src_v1), (dst_v0, dst_v1), (nrm_v0, nrm_v1)
        sems = (sem0, sem1)
        cid = lax.axis_index("c")
        sid = lax.axis_index("s")

        if True:
            if d == 4:
                cidx = 2 * (sid % ncpc) + cid
                ridx = sid // ncpc
            else:
                cidx = jnp.int32(0)
                ridx = cid * 16 + sid
            if rp == 1:
                plist = [(cidx & 1, cidx)]
            elif rp == 2:
                plist = [(jnp.int32(0), jnp.int32(0)),
                         (jnp.int32(1), jnp.int32(0))]
            else:
                plist = []
            _load_col_reduced(base, jnp.int32(boff) + cidx, part, plist,
                              in_col, tmps, sems)
            _zero_col(acc_col)
            ebase = ridx * er

            def issue(ci, slot):
                o = ebase + ci * chunk
                pltpu.async_copy(src.at[pl.ds(o, chunk)], svs[slot],
                                 sems[slot])
                pltpu.async_copy(dst.at[pl.ds(o, chunk)], dvs[slot],
                                 sems[slot])
                pltpu.async_copy(nrm.at[pl.ds(o, chunk)], nvs[slot],
                                 sems[slot])

            def drain(ci, slot):
                o = ebase + ci * chunk
                pltpu.make_async_copy(src.at[pl.ds(o, chunk)],
                                      svs[slot], sems[slot]).wait()
                pltpu.make_async_copy(dst.at[pl.ds(o, chunk)],
                                      dvs[slot], sems[slot]).wait()
                pltpu.make_async_copy(nrm.at[pl.ds(o, chunk)],
                                      nvs[slot], sems[slot]).wait()

            issue(0, 0)
            if nchunks > 1:
                issue(1, 1)
            for ci in range(nchunks):
                slot = ci % 2
                drain(ci, slot)
                sv, dv, nv = svs[slot], dvs[slot], nvs[slot]

                def vbody(j, _):
                    jb = j * (unroll * LANES)
                    for u in range(unroll):
                        s = pl.ds(jb + u * LANES, LANES)
                        si = sv[s]
                        di = dv[s]
                        w = nv[s]
                        gth = plsc.load_gather(in_col, [si])
                        plsc.addupdate_scatter(acc_col, [di], gth * w)
                    return jnp.int32(0)

                _fori(gpc // unroll, vbody)
                if ci + 2 < nchunks:
                    issue(ci + 2, slot)

            pltpu.sync_copy(acc_col, shared.at[sid])
            plsc.subcore_barrier()

            cl2 = sid % ncpc
            k = sid // ncpc
            soff = k * segw
            row0 = cl2
            pltpu.sync_copy(shared.at[row0, pl.ds(soff, segw)], seg_a)
            for p in range(1, nper):
                row = cl2 + ncpc * p
                pltpu.sync_copy(shared.at[row, pl.ds(soff, segw)], seg_b)

                def sbody(i, _):
                    b2 = i * (8 * LANES)
                    for u in range(8):
                        sl = pl.ds(b2 + u * LANES, LANES)
                        seg_a[sl] = seg_a[sl] + seg_b[sl]
                    return jnp.int32(0)

                _fori(segw // (8 * LANES), sbody)

            ocol = 2 * cl2 + cid if d == 4 else jnp.int32(0)
            pltpu.sync_copy(seg_a, out.at[cid, ocol, pl.ds(soff, segw)])

    return prop


_DEG_TILES = 20
_DEG_ER = E // _DEG_TILES
_DEG_CHUNK = 4000
_DEG_NC = _DEG_ER // _DEG_CHUNK


@functools.partial(
    pl.kernel,
    out_type=jax.ShapeDtypeStruct((_DEG_TILES, NP), jnp.float32),
    mesh=_MESH,
    compiler_params=_SC_PARAMS,
    scratch_types=[
        pltpu.VMEM((NP,), jnp.float32),
        pltpu.VMEM((_DEG_CHUNK,), jnp.int32),
        pltpu.VMEM((_DEG_CHUNK,), jnp.int32),
        pltpu.VMEM((_DEG_CHUNK,), jnp.float32),
        pltpu.VMEM((_DEG_CHUNK,), jnp.float32),
        pltpu.SemaphoreType.DMA,
        pltpu.SemaphoreType.DMA,
    ],
)
def _deg_kernel(dst, ew, out, acc_col, dst_v0, dst_v1, ew_v0, ew_v1,
                sem0, sem1):
    wid = _wid()
    dvs, wvs = (dst_v0, dst_v1), (ew_v0, ew_v1)
    sems = (sem0, sem1)

    @pl.when(wid < _DEG_TILES)
    def _():
        _zero_col(acc_col)
        ebase = wid * _DEG_ER

        def issue(ci, slot):
            o = ebase + ci * _DEG_CHUNK
            pltpu.async_copy(dst.at[pl.ds(o, _DEG_CHUNK)], dvs[slot],
                             sems[slot])
            pltpu.async_copy(ew.at[pl.ds(o, _DEG_CHUNK)], wvs[slot],
                             sems[slot])

        def drain(ci, slot):
            o = ebase + ci * _DEG_CHUNK
            pltpu.make_async_copy(dst.at[pl.ds(o, _DEG_CHUNK)],
                                  dvs[slot], sems[slot]).wait()
            pltpu.make_async_copy(ew.at[pl.ds(o, _DEG_CHUNK)],
                                  wvs[slot], sems[slot]).wait()

        issue(0, 0)
        issue(1, 1)
        for ci in range(_DEG_NC):
            slot = ci % 2
            drain(ci, slot)
            dv, wv = dvs[slot], wvs[slot]

            def vbody(j, _):
                base = j * (UNROLL * LANES)
                for u in range(UNROLL):
                    s = pl.ds(base + u * LANES, LANES)
                    plsc.addupdate_scatter(acc_col, [dv[s]], wv[s])
                return jnp.int32(0)

            _fori(_DEG_CHUNK // LANES // UNROLL, vbody)
            if ci + 2 < _DEG_NC:
                issue(ci + 2, slot)

        pltpu.sync_copy(acc_col, out.at[wid])


@functools.partial(
    pl.kernel,
    out_type=jax.ShapeDtypeStruct((E,), jnp.float32),
    mesh=_MESH,
    compiler_params=_SC_PARAMS,
    scratch_types=[
        pltpu.VMEM((NP,), jnp.float32),
        pltpu.VMEM((_DEG_CHUNK,), jnp.int32),
        pltpu.VMEM((_DEG_CHUNK,), jnp.int32),
        pltpu.VMEM((_DEG_CHUNK,), jnp.int32),
        pltpu.VMEM((_DEG_CHUNK,), jnp.int32),
        pltpu.VMEM((_DEG_CHUNK,), jnp.float32),
        pltpu.VMEM((_DEG_CHUNK,), jnp.float32),
        pltpu.VMEM((_DEG_CHUNK,), jnp.float32),
        pltpu.VMEM((_DEG_CHUNK,), jnp.float32),
        pltpu.SemaphoreType.DMA,
        pltpu.SemaphoreType.DMA,
        pltpu.SemaphoreType.DMA,
    ],
)
def _norm_kernel(dis, src, dst, ew, out, dis_col, src_v0, src_v1,
                 dst_v0, dst_v1, ew_v0, ew_v1, nrm_v0, nrm_v1,
                 sem0, sem1, sem_st):
    wid = _wid()
    svs, dvs = (src_v0, src_v1), (dst_v0, dst_v1)
    wvs, nvs = (ew_v0, ew_v1), (nrm_v0, nrm_v1)
    sems = (sem0, sem1)

    @pl.when(wid < _DEG_TILES)
    def _():
        pltpu.sync_copy(dis, dis_col)
        ebase = wid * _DEG_ER

        def issue(ci, slot):
            o = ebase + ci * _DEG_CHUNK
            pltpu.async_copy(src.at[pl.ds(o, _DEG_CHUNK)], svs[slot],
                             sems[slot])
            pltpu.async_copy(dst.at[pl.ds(o, _DEG_CHUNK)], dvs[slot],
                             sems[slot])
            pltpu.async_copy(ew.at[pl.ds(o, _DEG_CHUNK)], wvs[slot],
                             sems[slot])

        def drain(ci, slot):
            o = ebase + ci * _DEG_CHUNK
            pltpu.make_async_copy(src.at[pl.ds(o, _DEG_CHUNK)],
                                  svs[slot], sems[slot]).wait()
            pltpu.make_async_copy(dst.at[pl.ds(o, _DEG_CHUNK)],
                                  dvs[slot], sems[slot]).wait()
            pltpu.make_async_copy(ew.at[pl.ds(o, _DEG_CHUNK)],
                                  wvs[slot], sems[slot]).wait()

        def wait_store(ci, slot):
            o = ebase + ci * _DEG_CHUNK
            pltpu.make_async_copy(nvs[slot],
                                  out.at[pl.ds(o, _DEG_CHUNK)],
                                  sem_st).wait()

        issue(0, 0)
        issue(1, 1)
        for ci in range(_DEG_NC):
            slot = ci % 2
            drain(ci, slot)
            if ci >= 2:
                wait_store(ci - 2, slot)
            sv, dv, wv, nv = svs[slot], dvs[slot], wvs[slot], nvs[slot]

            def vbody(j, _):
                base = j * (UNROLL * LANES)
                for u in range(UNROLL):
                    s = pl.ds(base + u * LANES, LANES)
                    g1 = plsc.load_gather(dis_col, [sv[s]])
                    g2 = plsc.load_gather(dis_col, [dv[s]])
                    nv[s] = g1 * wv[s] * g2
                return jnp.int32(0)

            _fori(_DEG_CHUNK // LANES // UNROLL, vbody)
            o = ebase + ci * _DEG_CHUNK
            pltpu.async_copy(nvs[slot], out.at[pl.ds(o, _DEG_CHUNK)],
                             sem_st)
            if ci + 2 < _DEG_NC:
                issue(ci + 2, slot)

        for ci in range(max(_DEG_NC - 2, 0), _DEG_NC):
            wait_store(ci, ci % 2)


def _proj1_body(x_ref, w_ref, out_ref):
    out_ref[...] = lax.dot_general(
        w_ref[...], x_ref[...], (((0,), (1,)), ((), ())),
        preferred_element_type=jnp.float32)


def _dis_body(degp_ref, out_ref):
    deg = jnp.sum(degp_ref[...], axis=0, keepdims=True)
    out_ref[...] = jnp.where(deg > 0.0, lax.rsqrt(deg), 0.0)


def _mid_body(y0_ref, t3_ref, sel_ref, b1_ref, w2_ref, out_ref):
    t3 = jnp.where(sel_ref[...] > 0.0, t3_ref[1], t3_ref[0])
    h = y0_ref[...] + t3 + b1_ref[...]
    h = jnp.maximum(h, 0.0)
    out_ref[...] = lax.dot_general(
        w2_ref[...], h, (((1,), (0,)), ((), ())),
        preferred_element_type=jnp.float32)


def _fin_body(v0_ref, q_ref, b2_ref, wl_ref, bl_ref, out_ref):
    o = (v0_ref[...] + jnp.sum(q_ref[...], axis=0, keepdims=True)
         + b2_ref[...])
    o = o * wl_ref[...] + bl_ref[...]
    out_ref[...] = 1.0 / (1.0 + jnp.exp(-o))


_P1 = (4, 12, 0, 16, 8, 4000)
_P2 = (4, 8, 1, 16, 8, 4000)
_P3 = (4, 4, 1, 16, 8, 4000)
_P4 = (1, 3, 0, 4, 32, 2000)
_P5 = (1, 2, 2, 4, 32, 2000)
_P6 = (1, 1, 2, 4, 32, 2000)


def kernel(x, edge_index, edge_attr, W1, b1, W2, b2, Wl, bl):
    src = edge_index[0].astype(jnp.int32)
    dst = edge_index[1].astype(jnp.int32)
    ew = edge_attr.astype(jnp.float32)
    xpad = jnp.zeros((NP, F_IN), jnp.float32).at[:N].set(
        x.astype(jnp.float32))
    wcat1 = jnp.concatenate([W1[0], W1[1], W1[2], W1[3]],
                            axis=1).astype(jnp.float32)

    yt = pl.pallas_call(
        _proj1_body,
        out_shape=jax.ShapeDtypeStruct((16, NP), jnp.float32),
    )(xpad, wcat1)

    degp = _deg_kernel(dst, ew)
    dis = pl.pallas_call(
        _dis_body,
        out_shape=jax.ShapeDtypeStruct((1, NP), jnp.float32),
    )(degp)

    nrm = _norm_kernel(dis.reshape(NP), src, dst, ew)

    t = _make_prop(*_P1)(yt, src, dst, nrm)
    t = _make_prop(*_P2)(yt, t, src, dst, nrm)
    t = _make_prop(*_P3)(yt, t, src, dst, nrm)

    w2mat = W2[:, :, 0].astype(jnp.float32)
    sel = (jnp.arange(4, dtype=jnp.float32) % 2).reshape(4, 1)
    v = pl.pallas_call(
        _mid_body,
        out_shape=jax.ShapeDtypeStruct((4, NP), jnp.float32),
    )(yt[0:4], t, sel, b1.reshape(4, 1), w2mat)

    q = _make_prop(*_P4)(v, src, dst, nrm)
    q = _make_prop(*_P5)(v, q, src, dst, nrm)
    q = _make_prop(*_P6)(v, q, src, dst, nrm)

    o = pl.pallas_call(
        _fin_body,
        out_shape=jax.ShapeDtypeStruct((1, NP), jnp.float32),
    )(v[0:1], q[:, 0],
      b2.reshape(1, 1).astype(jnp.float32), Wl.astype(jnp.float32),
      bl.reshape(1, 1).astype(jnp.float32))

    return o[0, :N].reshape(N, 1).astype(jnp.float64)


# speedup vs baseline: 699.2669x; 1.2652x over previous
import functools

import jax
import jax.numpy as jnp
from jax import lax
from jax.experimental import pallas as pl
from jax.experimental.pallas import tpu as pltpu
from jax.experimental.pallas import tpu_sc as plsc

N = 10000
NP = 10240
E = 320000
F_IN = 128
LANES = 16
UNROLL = 10

_MESH = plsc.VectorSubcoreMesh(core_axis_name="c", subcore_axis_name="s")
_SC_PARAMS = pltpu.CompilerParams(needs_layout_passes=False)
_NTILES = 32


def _wid():
    return lax.axis_index("s") * 2 + lax.axis_index("c")


def _fori(n, body):
    lax.fori_loop(jnp.int32(0), jnp.int32(n), body, jnp.int32(0))


def _zero_col(col):
    z = jnp.zeros((LANES,), jnp.float32)

    def body(i, _):
        base = i * (8 * LANES)
        for u in range(8):
            col[pl.ds(base + u * LANES, LANES)] = z
        return jnp.int32(0)

    _fori(NP // (8 * LANES), body)


def _add_into(dst_col, src_col):
    def body(i, _):
        base = i * (8 * LANES)
        for u in range(8):
            s = pl.ds(base + u * LANES, LANES)
            dst_col[s] = dst_col[s] + src_col[s]
        return jnp.int32(0)

    _fori(NP // (8 * LANES), body)


def _load_col_reduced(base, bcol, part, plist, in_col, tmps, sems):
    pltpu.async_copy(base.at[bcol], in_col, sems[0])
    pltpu.make_async_copy(base.at[bcol], in_col, sems[0]).wait()
    for i, (pi, pc) in enumerate(plist):
        if i == 0:
            pltpu.async_copy(part.at[pi, pc], tmps[0], sems[0])
        if i + 1 < len(plist):
            pi2, pc2 = plist[i + 1]
            s2 = (i + 1) % 2
            pltpu.async_copy(part.at[pi2, pc2], tmps[s2], sems[s2])
        sl = i % 2
        pltpu.make_async_copy(part.at[pi, pc], tmps[sl], sems[sl]).wait()
        _add_into(in_col, tmps[sl])


def _make_prop(d, boff, rp, base_d, nranges, chunk):
    assert d in (1, 4)
    ncpc = max(d // 2, 1)
    nranges = 8 if d == 4 else 32
    er = E // nranges
    nchunks = er // chunk
    gpc = chunk // LANES
    unroll = UNROLL if gpc % UNROLL == 0 else 5
    nper = 16 // ncpc
    segw = NP // nper
    assert er % chunk == 0 and gpc % unroll == 0 and NP % nper == 0

    @functools.partial(
        pl.kernel,
        out_type=jax.ShapeDtypeStruct((2, d, NP), jnp.float32),
        mesh=_MESH,
        compiler_params=_SC_PARAMS,
        scratch_types=(
            [pltpu.VMEM((NP,), jnp.float32) for _ in range(4)]
            + [pltpu.VMEM_SHARED((16, NP), jnp.float32),
               pltpu.VMEM((segw,), jnp.float32),
               pltpu.VMEM((segw,), jnp.float32),
               pltpu.VMEM((chunk,), jnp.int32),
               pltpu.VMEM((chunk,), jnp.int32),
               pltpu.VMEM((chunk,), jnp.int32),
               pltpu.VMEM((chunk,), jnp.int32),
               pltpu.VMEM((chunk,), jnp.float32),
               pltpu.VMEM((chunk,), jnp.float32),
               pltpu.SemaphoreType.DMA,
               pltpu.SemaphoreType.DMA]
        ),
    )
    def prop(*args):
        if rp:
            (base, part, src, dst, nrm, out, in_col, acc_col, tmp0, tmp1,
             shared, seg_a, seg_b,
             src_v0, src_v1, dst_v0, dst_v1, nrm_v0, nrm_v1,
             sem0, sem1) = args
        else:
            (base, src, dst, nrm, out, in_col, acc_col, tmp0, tmp1,
             shared, seg_a, seg_b,
             src_v0, src_v1, dst_v0, dst_v1, nrm_v0, nrm_v1,
             sem0, sem1) = args
            part = None
        tmps = (tmp0, tmp1)
        svs, dvs, nvs = (src_v0, src_v1), (dst_v0, dst_v1), (nrm_v0, nrm_v1)
        sems = (sem0, sem1)
        cid = lax.axis_index("c")
        sid = lax.axis_index("s")

        if True:
            if d == 4:
                cidx = 2 * (sid % ncpc) + cid
                ridx = sid // ncpc
            else:
                cidx = jnp.int32(0)
                ridx = cid * 16 + sid
            if rp == 1:
                plist = [(cidx & 1, cidx)]
            elif rp == 2:
                plist = [(jnp.int32(0), jnp.int32(0)),
                         (jnp.int32(1), jnp.int32(0))]
            else:
                plist = []
            _load_col_reduced(base, jnp.int32(boff) + cidx, part, plist,
                              in_col, tmps, sems)
            _zero_col(acc_col)
            ebase = ridx * er

            def issue(ci, slot):
                o = ebase + ci * chunk
                pltpu.async_copy(src.at[pl.ds(o, chunk)], svs[slot],
                                 sems[slot])
                pltpu.async_copy(dst.at[pl.ds(o, chunk)], dvs[slot],
                                 sems[slot])
                pltpu.async_copy(nrm.at[pl.ds(o, chunk)], nvs[slot],
                                 sems[slot])

            def drain(ci, slot):
                o = ebase + ci * chunk
                pltpu.make_async_copy(src.at[pl.ds(o, chunk)],
                                      svs[slot], sems[slot]).wait()
                pltpu.make_async_copy(dst.at[pl.ds(o, chunk)],
                                      dvs[slot], sems[slot]).wait()
                pltpu.make_async_copy(nrm.at[pl.ds(o, chunk)],
                                      nvs[slot], sems[slot]).wait()

            issue(0, 0)
            if nchunks > 1:
                issue(1, 1)
            for ci in range(nchunks):
                slot = ci % 2
                drain(ci, slot)
                sv, dv, nv = svs[slot], dvs[slot], nvs[slot]

                @plsc.parallel_loop(jnp.int32(0), jnp.int32(gpc),
                                    jnp.int32(1), unroll=unroll)
                def vbody(j):
                    s = pl.ds(j * LANES, LANES)
                    si = sv[s]
                    di = dv[s]
                    w = nv[s]
                    gth = plsc.load_gather(in_col, [si])
                    plsc.addupdate_scatter(acc_col, [di], gth * w)
                if ci + 2 < nchunks:
                    issue(ci + 2, slot)

            pltpu.sync_copy(acc_col, shared.at[sid])
            plsc.subcore_barrier()

            cl2 = sid % ncpc
            k = sid // ncpc
            soff = k * segw
            row0 = cl2
            pltpu.sync_copy(shared.at[row0, pl.ds(soff, segw)], seg_a)
            for p in range(1, nper):
                row = cl2 + ncpc * p
                pltpu.sync_copy(shared.at[row, pl.ds(soff, segw)], seg_b)

                def sbody(i, _):
                    b2 = i * (8 * LANES)
                    for u in range(8):
                        sl = pl.ds(b2 + u * LANES, LANES)
                        seg_a[sl] = seg_a[sl] + seg_b[sl]
                    return jnp.int32(0)

                _fori(segw // (8 * LANES), sbody)

            ocol = 2 * cl2 + cid if d == 4 else jnp.int32(0)
            pltpu.sync_copy(seg_a, out.at[cid, ocol, pl.ds(soff, segw)])

    return prop


_DEG_TILES = 20
_DEG_ER = E // _DEG_TILES
_DEG_CHUNK = 4000
_DEG_NC = _DEG_ER // _DEG_CHUNK


@functools.partial(
    pl.kernel,
    out_type=jax.ShapeDtypeStruct((_DEG_TILES, NP), jnp.float32),
    mesh=_MESH,
    compiler_params=_SC_PARAMS,
    scratch_types=[
        pltpu.VMEM((NP,), jnp.float32),
        pltpu.VMEM((_DEG_CHUNK,), jnp.int32),
        pltpu.VMEM((_DEG_CHUNK,), jnp.int32),
        pltpu.VMEM((_DEG_CHUNK,), jnp.float32),
        pltpu.VMEM((_DEG_CHUNK,), jnp.float32),
        pltpu.SemaphoreType.DMA,
        pltpu.SemaphoreType.DMA,
    ],
)
def _deg_kernel(dst, ew, out, acc_col, dst_v0, dst_v1, ew_v0, ew_v1,
                sem0, sem1):
    wid = _wid()
    dvs, wvs = (dst_v0, dst_v1), (ew_v0, ew_v1)
    sems = (sem0, sem1)

    @pl.when(wid < _DEG_TILES)
    def _():
        _zero_col(acc_col)
        ebase = wid * _DEG_ER

        def issue(ci, slot):
            o = ebase + ci * _DEG_CHUNK
            pltpu.async_copy(dst.at[pl.ds(o, _DEG_CHUNK)], dvs[slot],
                             sems[slot])
            pltpu.async_copy(ew.at[pl.ds(o, _DEG_CHUNK)], wvs[slot],
                             sems[slot])

        def drain(ci, slot):
            o = ebase + ci * _DEG_CHUNK
            pltpu.make_async_copy(dst.at[pl.ds(o, _DEG_CHUNK)],
                                  dvs[slot], sems[slot]).wait()
            pltpu.make_async_copy(ew.at[pl.ds(o, _DEG_CHUNK)],
                                  wvs[slot], sems[slot]).wait()

        issue(0, 0)
        issue(1, 1)
        for ci in range(_DEG_NC):
            slot = ci % 2
            drain(ci, slot)
            dv, wv = dvs[slot], wvs[slot]

            def vbody(j, _):
                base = j * (UNROLL * LANES)
                for u in range(UNROLL):
                    s = pl.ds(base + u * LANES, LANES)
                    plsc.addupdate_scatter(acc_col, [dv[s]], wv[s])
                return jnp.int32(0)

            _fori(_DEG_CHUNK // LANES // UNROLL, vbody)
            if ci + 2 < _DEG_NC:
                issue(ci + 2, slot)

        pltpu.sync_copy(acc_col, out.at[wid])


@functools.partial(
    pl.kernel,
    out_type=jax.ShapeDtypeStruct((E,), jnp.float32),
    mesh=_MESH,
    compiler_params=_SC_PARAMS,
    scratch_types=[
        pltpu.VMEM((NP,), jnp.float32),
        pltpu.VMEM((_DEG_CHUNK,), jnp.int32),
        pltpu.VMEM((_DEG_CHUNK,), jnp.int32),
        pltpu.VMEM((_DEG_CHUNK,), jnp.int32),
        pltpu.VMEM((_DEG_CHUNK,), jnp.int32),
        pltpu.VMEM((_DEG_CHUNK,), jnp.float32),
        pltpu.VMEM((_DEG_CHUNK,), jnp.float32),
        pltpu.VMEM((_DEG_CHUNK,), jnp.float32),
        pltpu.VMEM((_DEG_CHUNK,), jnp.float32),
        pltpu.SemaphoreType.DMA,
        pltpu.SemaphoreType.DMA,
        pltpu.SemaphoreType.DMA,
    ],
)
def _norm_kernel(dis, src, dst, ew, out, dis_col, src_v0, src_v1,
                 dst_v0, dst_v1, ew_v0, ew_v1, nrm_v0, nrm_v1,
                 sem0, sem1, sem_st):
    wid = _wid()
    svs, dvs = (src_v0, src_v1), (dst_v0, dst_v1)
    wvs, nvs = (ew_v0, ew_v1), (nrm_v0, nrm_v1)
    sems = (sem0, sem1)

    @pl.when(wid < _DEG_TILES)
    def _():
        pltpu.sync_copy(dis, dis_col)
        ebase = wid * _DEG_ER

        def issue(ci, slot):
            o = ebase + ci * _DEG_CHUNK
            pltpu.async_copy(src.at[pl.ds(o, _DEG_CHUNK)], svs[slot],
                             sems[slot])
            pltpu.async_copy(dst.at[pl.ds(o, _DEG_CHUNK)], dvs[slot],
                             sems[slot])
            pltpu.async_copy(ew.at[pl.ds(o, _DEG_CHUNK)], wvs[slot],
                             sems[slot])

        def drain(ci, slot):
            o = ebase + ci * _DEG_CHUNK
            pltpu.make_async_copy(src.at[pl.ds(o, _DEG_CHUNK)],
                                  svs[slot], sems[slot]).wait()
            pltpu.make_async_copy(dst.at[pl.ds(o, _DEG_CHUNK)],
                                  dvs[slot], sems[slot]).wait()
            pltpu.make_async_copy(ew.at[pl.ds(o, _DEG_CHUNK)],
                                  wvs[slot], sems[slot]).wait()

        def wait_store(ci, slot):
            o = ebase + ci * _DEG_CHUNK
            pltpu.make_async_copy(nvs[slot],
                                  out.at[pl.ds(o, _DEG_CHUNK)],
                                  sem_st).wait()

        issue(0, 0)
        issue(1, 1)
        for ci in range(_DEG_NC):
            slot = ci % 2
            drain(ci, slot)
            if ci >= 2:
                wait_store(ci - 2, slot)
            sv, dv, wv, nv = svs[slot], dvs[slot], wvs[slot], nvs[slot]

            def vbody(j, _):
                base = j * (UNROLL * LANES)
                for u in range(UNROLL):
                    s = pl.ds(base + u * LANES, LANES)
                    g1 = plsc.load_gather(dis_col, [sv[s]])
                    g2 = plsc.load_gather(dis_col, [dv[s]])
                    nv[s] = g1 * wv[s] * g2
                return jnp.int32(0)

            _fori(_DEG_CHUNK // LANES // UNROLL, vbody)
            o = ebase + ci * _DEG_CHUNK
            pltpu.async_copy(nvs[slot], out.at[pl.ds(o, _DEG_CHUNK)],
                             sem_st)
            if ci + 2 < _DEG_NC:
                issue(ci + 2, slot)

        for ci in range(max(_DEG_NC - 2, 0), _DEG_NC):
            wait_store(ci, ci % 2)


def _proj1_body(x_ref, w_ref, out_ref):
    out_ref[...] = lax.dot_general(
        w_ref[...], x_ref[...], (((0,), (1,)), ((), ())),
        preferred_element_type=jnp.float32)


def _dis_body(degp_ref, out_ref):
    deg = jnp.sum(degp_ref[...], axis=0, keepdims=True)
    out_ref[...] = jnp.where(deg > 0.0, lax.rsqrt(deg), 0.0)


def _mid_body(y0_ref, t3_ref, sel_ref, b1_ref, w2_ref, out_ref):
    t3 = jnp.where(sel_ref[...] > 0.0, t3_ref[1], t3_ref[0])
    h = y0_ref[...] + t3 + b1_ref[...]
    h = jnp.maximum(h, 0.0)
    out_ref[...] = lax.dot_general(
        w2_ref[...], h, (((1,), (0,)), ((), ())),
        preferred_element_type=jnp.float32)


def _fin_body(v0_ref, q_ref, b2_ref, wl_ref, bl_ref, out_ref):
    o = (v0_ref[...] + jnp.sum(q_ref[...], axis=0, keepdims=True)
         + b2_ref[...])
    o = o * wl_ref[...] + bl_ref[...]
    out_ref[...] = 1.0 / (1.0 + jnp.exp(-o))


_P1 = (4, 12, 0, 16, 8, 4000)
_P2 = (4, 8, 1, 16, 8, 4000)
_P3 = (4, 4, 1, 16, 8, 4000)
_P4 = (1, 3, 0, 4, 32, 2000)
_P5 = (1, 2, 2, 4, 32, 2000)
_P6 = (1, 1, 2, 4, 32, 2000)


def kernel(x, edge_index, edge_attr, W1, b1, W2, b2, Wl, bl):
    src = edge_index[0].astype(jnp.int32)
    dst = edge_index[1].astype(jnp.int32)
    ew = edge_attr.astype(jnp.float32)
    xpad = jnp.zeros((NP, F_IN), jnp.float32).at[:N].set(
        x.astype(jnp.float32))
    wcat1 = jnp.concatenate([W1[0], W1[1], W1[2], W1[3]],
                            axis=1).astype(jnp.float32)

    yt = pl.pallas_call(
        _proj1_body,
        out_shape=jax.ShapeDtypeStruct((16, NP), jnp.float32),
    )(xpad, wcat1)

    degp = _deg_kernel(dst, ew)
    dis = pl.pallas_call(
        _dis_body,
        out_shape=jax.ShapeDtypeStruct((1, NP), jnp.float32),
    )(degp)

    nrm = _norm_kernel(dis.reshape(NP), src, dst, ew)

    t = _make_prop(*_P1)(yt, src, dst, nrm)
    t = _make_prop(*_P2)(yt, t, src, dst, nrm)
    t = _make_prop(*_P3)(yt, t, src, dst, nrm)

    w2mat = W2[:, :, 0].astype(jnp.float32)
    sel = (jnp.arange(4, dtype=jnp.float32) % 2).reshape(4, 1)
    v = pl.pallas_call(
        _mid_body,
        out_shape=jax.ShapeDtypeStruct((4, NP), jnp.float32),
    )(yt[0:4], t, sel, b1.reshape(4, 1), w2mat)

    q = _make_prop(*_P4)(v, src, dst, nrm)
    q = _make_prop(*_P5)(v, q, src, dst, nrm)
    q = _make_prop(*_P6)(v, q, src, dst, nrm)

    o = pl.pallas_call(
        _fin_body,
        out_shape=jax.ShapeDtypeStruct((1, NP), jnp.float32),
    )(v[0:1], q[:, 0],
      b2.reshape(1, 1).astype(jnp.float32), Wl.astype(jnp.float32),
      bl.reshape(1, 1).astype(jnp.float32))

    return o[0, :N].reshape(N, 1).astype(jnp.float64)


# speedup vs baseline: 759.0140x; 1.0854x over previous
import functools

import jax
import jax.numpy as jnp
from jax import lax
from jax.experimental import pallas as pl
from jax.experimental.pallas import tpu as pltpu
from jax.experimental.pallas import tpu_sc as plsc

N = 10000
NP = 10240
E = 320000
F_IN = 128
LANES = 16
UNROLL = 10

_MESH = plsc.VectorSubcoreMesh(core_axis_name="c", subcore_axis_name="s")
_SC_PARAMS = pltpu.CompilerParams(needs_layout_passes=False)
_NTILES = 32


def _wid():
    return lax.axis_index("s") * 2 + lax.axis_index("c")


def _fori(n, body):
    lax.fori_loop(jnp.int32(0), jnp.int32(n), body, jnp.int32(0))


def _zero_col(col):
    z = jnp.zeros((LANES,), jnp.float32)

    def body(i, _):
        base = i * (8 * LANES)
        for u in range(8):
            col[pl.ds(base + u * LANES, LANES)] = z
        return jnp.int32(0)

    _fori(NP // (8 * LANES), body)


def _add_into(dst_col, src_col):
    def body(i, _):
        base = i * (8 * LANES)
        for u in range(8):
            s = pl.ds(base + u * LANES, LANES)
            dst_col[s] = dst_col[s] + src_col[s]
        return jnp.int32(0)

    _fori(NP // (8 * LANES), body)


def _load_col_reduced(base, bcol, part, plist, in_col, tmps, sems):
    pltpu.async_copy(base.at[bcol], in_col, sems[0])
    pltpu.make_async_copy(base.at[bcol], in_col, sems[0]).wait()
    for i, (pi, pc) in enumerate(plist):
        if i == 0:
            pltpu.async_copy(part.at[pi, pc], tmps[0], sems[0])
        if i + 1 < len(plist):
            pi2, pc2 = plist[i + 1]
            s2 = (i + 1) % 2
            pltpu.async_copy(part.at[pi2, pc2], tmps[s2], sems[s2])
        sl = i % 2
        pltpu.make_async_copy(part.at[pi, pc], tmps[sl], sems[sl]).wait()
        _add_into(in_col, tmps[sl])


def _make_prop(d, boff, rp, base_d, nranges, chunk):
    assert d in (1, 4)
    ncpc = max(d // 2, 1)
    nranges = 8 if d == 4 else 32
    er = E // nranges
    nchunks = er // chunk
    gpc = chunk // LANES
    unroll = UNROLL if gpc % UNROLL == 0 else 5
    nper = 16 // ncpc
    segw = NP // nper
    assert er % chunk == 0 and gpc % unroll == 0 and NP % nper == 0

    @functools.partial(
        pl.kernel,
        out_type=jax.ShapeDtypeStruct((2, d, NP), jnp.float32),
        mesh=_MESH,
        compiler_params=_SC_PARAMS,
        scratch_types=(
            [pltpu.VMEM((NP,), jnp.float32) for _ in range(4)]
            + [pltpu.VMEM_SHARED((16, NP), jnp.float32),
               pltpu.VMEM((segw,), jnp.float32),
               pltpu.VMEM((segw,), jnp.float32),
               pltpu.VMEM((chunk,), jnp.int32),
               pltpu.VMEM((chunk,), jnp.int32),
               pltpu.VMEM((chunk,), jnp.int32),
               pltpu.VMEM((chunk,), jnp.int32),
               pltpu.VMEM((chunk,), jnp.float32),
               pltpu.VMEM((chunk,), jnp.float32),
               pltpu.SemaphoreType.DMA,
               pltpu.SemaphoreType.DMA]
        ),
    )
    def prop(*args):
        if rp:
            (base, part, src, dst, nrm, out, in_col, acc_col, tmp0, tmp1,
             shared, seg_a, seg_b,
             src_v0, src_v1, dst_v0, dst_v1, nrm_v0, nrm_v1,
             sem0, sem1) = args
        else:
            (base, src, dst, nrm, out, in_col, acc_col, tmp0, tmp1,
             shared, seg_a, seg_b,
             src_v0, src_v1, dst_v0, dst_v1, nrm_v0, nrm_v1,
             sem0, sem1) = args
            part = None
        tmps = (tmp0, tmp1)
        svs, dvs, nvs = (src_v0, src_v1), (dst_v0, dst_v1), (nrm_v0, nrm_v1)
        sems = (sem0, sem1)
        cid = lax.axis_index("c")
        sid = lax.axis_index("s")

        if True:
            if d == 4:
                cidx = 2 * (sid % ncpc) + cid
                ridx = sid // ncpc
            else:
                cidx = jnp.int32(0)
                ridx = cid * 16 + sid
            if rp == 1:
                plist = [(cidx & 1, cidx)]
            elif rp == 2:
                plist = [(jnp.int32(0), jnp.int32(0)),
                         (jnp.int32(1), jnp.int32(0))]
            else:
                plist = []
            _load_col_reduced(base, jnp.int32(boff) + cidx, part, plist,
                              in_col, tmps, sems)
            _zero_col(acc_col)
            ebase = ridx * er

            def issue(ci, slot):
                o = ebase + ci * chunk
                pltpu.async_copy(src.at[pl.ds(o, chunk)], svs[slot],
                                 sems[slot])
                pltpu.async_copy(dst.at[pl.ds(o, chunk)], dvs[slot],
                                 sems[slot])
                pltpu.async_copy(nrm.at[pl.ds(o, chunk)], nvs[slot],
                                 sems[slot])

            def drain(ci, slot):
                o = ebase + ci * chunk
                pltpu.make_async_copy(src.at[pl.ds(o, chunk)],
                                      svs[slot], sems[slot]).wait()
                pltpu.make_async_copy(dst.at[pl.ds(o, chunk)],
                                      dvs[slot], sems[slot]).wait()
                pltpu.make_async_copy(nrm.at[pl.ds(o, chunk)],
                                      nvs[slot], sems[slot]).wait()

            issue(0, 0)
            if nchunks > 1:
                issue(1, 1)
            for ci in range(nchunks):
                slot = ci % 2
                drain(ci, slot)
                sv, dv, nv = svs[slot], dvs[slot], nvs[slot]

                @plsc.parallel_loop(jnp.int32(0), jnp.int32(gpc),
                                    jnp.int32(1), unroll=unroll)
                def vbody(j):
                    s = pl.ds(j * LANES, LANES)
                    si = sv[s]
                    di = dv[s]
                    w = nv[s]
                    gth = plsc.load_gather(in_col, [si])
                    plsc.addupdate_scatter(acc_col, [di], gth * w)
                if ci + 2 < nchunks:
                    issue(ci + 2, slot)

            pltpu.sync_copy(acc_col, shared.at[sid])
            plsc.subcore_barrier()

            cl2 = sid % ncpc
            k = sid // ncpc
            soff = k * segw
            row0 = cl2
            pltpu.sync_copy(shared.at[row0, pl.ds(soff, segw)], seg_a)
            for p in range(1, nper):
                row = cl2 + ncpc * p
                pltpu.sync_copy(shared.at[row, pl.ds(soff, segw)], seg_b)

                def sbody(i, _):
                    b2 = i * (8 * LANES)
                    for u in range(8):
                        sl = pl.ds(b2 + u * LANES, LANES)
                        seg_a[sl] = seg_a[sl] + seg_b[sl]
                    return jnp.int32(0)

                _fori(segw // (8 * LANES), sbody)

            ocol = 2 * cl2 + cid if d == 4 else jnp.int32(0)
            pltpu.sync_copy(seg_a, out.at[cid, ocol, pl.ds(soff, segw)])

    return prop


_PREP_DEG_ER = E // 16
_PREP_DEG_CHUNK = 4000
_PREP_DEG_NC = _PREP_DEG_ER // _PREP_DEG_CHUNK
_PREP_NRM_ER = E // 32
_PREP_NRM_CHUNK = 2000
_PREP_NRM_NC = _PREP_NRM_ER // _PREP_NRM_CHUNK
_SEGW = NP // 16


@functools.partial(
    pl.kernel,
    out_type=jax.ShapeDtypeStruct((E,), jnp.float32),
    mesh=_MESH,
    compiler_params=_SC_PARAMS,
    scratch_types=[
        pltpu.VMEM((NP,), jnp.float32),
        pltpu.VMEM_SHARED((16, NP), jnp.float32),
        pltpu.VMEM_SHARED((NP,), jnp.float32),
        pltpu.VMEM((_SEGW,), jnp.float32),
        pltpu.VMEM((_SEGW,), jnp.float32),
        pltpu.VMEM((_PREP_DEG_CHUNK,), jnp.int32),
        pltpu.VMEM((_PREP_DEG_CHUNK,), jnp.int32),
        pltpu.VMEM((_PREP_DEG_CHUNK,), jnp.float32),
        pltpu.VMEM((_PREP_DEG_CHUNK,), jnp.float32),
        pltpu.VMEM((_PREP_NRM_CHUNK,), jnp.int32),
        pltpu.VMEM((_PREP_NRM_CHUNK,), jnp.int32),
        pltpu.VMEM((_PREP_NRM_CHUNK,), jnp.int32),
        pltpu.VMEM((_PREP_NRM_CHUNK,), jnp.int32),
        pltpu.VMEM((_PREP_NRM_CHUNK,), jnp.float32),
        pltpu.VMEM((_PREP_NRM_CHUNK,), jnp.float32),
        pltpu.VMEM((_PREP_NRM_CHUNK,), jnp.float32),
        pltpu.VMEM((_PREP_NRM_CHUNK,), jnp.float32),
        pltpu.SemaphoreType.DMA,
        pltpu.SemaphoreType.DMA,
        pltpu.SemaphoreType.DMA,
    ],
)
def _prep_kernel(src, dst, ew, out, acc_col, shared, shared_dis, seg_a,
                 seg_b, d1_v0, d1_v1, w1_v0, w1_v1, s3_v0, s3_v1,
                 d3_v0, d3_v1, w3_v0, w3_v1, n3_v0, n3_v1,
                 sem0, sem1, sem_st):
    cid = lax.axis_index("c")
    sid = lax.axis_index("s")
    sems = (sem0, sem1)

    _zero_col(acc_col)
    dvs, wvs = (d1_v0, d1_v1), (w1_v0, w1_v1)
    ebase = sid * _PREP_DEG_ER

    def issue1(ci, slot):
        o = ebase + ci * _PREP_DEG_CHUNK
        pltpu.async_copy(dst.at[pl.ds(o, _PREP_DEG_CHUNK)], dvs[slot],
                         sems[slot])
        pltpu.async_copy(ew.at[pl.ds(o, _PREP_DEG_CHUNK)], wvs[slot],
                         sems[slot])

    def drain1(ci, slot):
        o = ebase + ci * _PREP_DEG_CHUNK
        pltpu.make_async_copy(dst.at[pl.ds(o, _PREP_DEG_CHUNK)], dvs[slot],
                              sems[slot]).wait()
        pltpu.make_async_copy(ew.at[pl.ds(o, _PREP_DEG_CHUNK)], wvs[slot],
                              sems[slot]).wait()

    issue1(0, 0)
    issue1(1, 1)
    for ci in range(_PREP_DEG_NC):
        slot = ci % 2
        drain1(ci, slot)
        dv, wv = dvs[slot], wvs[slot]

        @plsc.parallel_loop(jnp.int32(0), jnp.int32(_PREP_DEG_CHUNK // LANES),
                            jnp.int32(1), unroll=UNROLL)
        def vbody(j):
            sl = pl.ds(j * LANES, LANES)
            plsc.addupdate_scatter(acc_col, [dv[sl]], wv[sl])

        if ci + 2 < _PREP_DEG_NC:
            issue1(ci + 2, slot)

    pltpu.sync_copy(acc_col, shared.at[sid])
    plsc.subcore_barrier()

    soff = sid * _SEGW
    pltpu.sync_copy(shared.at[jnp.int32(0), pl.ds(soff, _SEGW)], seg_a)
    for p in range(1, 16):
        pltpu.sync_copy(shared.at[jnp.int32(p), pl.ds(soff, _SEGW)], seg_b)

        def sbody(i, _):
            b2 = i * (8 * LANES)
            for u in range(8):
                sl = pl.ds(b2 + u * LANES, LANES)
                seg_a[sl] = seg_a[sl] + seg_b[sl]
            return jnp.int32(0)

        _fori(_SEGW // (8 * LANES), sbody)

    def rbody(i, _):
        sl = pl.ds(i * LANES, LANES)
        d = seg_a[sl]
        bits = plsc.bitcast(d, jnp.int32)
        y = plsc.bitcast(jnp.int32(0x5F3759DF) -
                         lax.shift_right_logical(bits, jnp.int32(1)),
                         jnp.float32)
        hd = d * 0.5
        for _unused in range(3):
            y = y * (1.5 - hd * y * y)
        seg_a[sl] = jnp.where(d > 0.0, y, 0.0)
        return jnp.int32(0)

    _fori(_SEGW // LANES, rbody)
    pltpu.sync_copy(seg_a, shared_dis.at[pl.ds(soff, _SEGW)])
    plsc.subcore_barrier()

    pltpu.sync_copy(shared_dis, acc_col)
    svs, dvs3 = (s3_v0, s3_v1), (d3_v0, d3_v1)
    wvs3, nvs = (w3_v0, w3_v1), (n3_v0, n3_v1)
    nbase = (cid * 16 + sid) * _PREP_NRM_ER

    def issue3(ci, slot):
        o = nbase + ci * _PREP_NRM_CHUNK
        pltpu.async_copy(src.at[pl.ds(o, _PREP_NRM_CHUNK)], svs[slot],
                         sems[slot])
        pltpu.async_copy(dst.at[pl.ds(o, _PREP_NRM_CHUNK)], dvs3[slot],
                         sems[slot])
        pltpu.async_copy(ew.at[pl.ds(o, _PREP_NRM_CHUNK)], wvs3[slot],
                         sems[slot])

    def drain3(ci, slot):
        o = nbase + ci * _PREP_NRM_CHUNK
        pltpu.make_async_copy(src.at[pl.ds(o, _PREP_NRM_CHUNK)], svs[slot],
                              sems[slot]).wait()
        pltpu.make_async_copy(dst.at[pl.ds(o, _PREP_NRM_CHUNK)], dvs3[slot],
                              sems[slot]).wait()
        pltpu.make_async_copy(ew.at[pl.ds(o, _PREP_NRM_CHUNK)], wvs3[slot],
                              sems[slot]).wait()

    def wait_store(ci, slot):
        o = nbase + ci * _PREP_NRM_CHUNK
        pltpu.make_async_copy(nvs[slot], out.at[pl.ds(o, _PREP_NRM_CHUNK)],
                              sem_st).wait()

    issue3(0, 0)
    issue3(1, 1)
    for ci in range(_PREP_NRM_NC):
        slot = ci % 2
        drain3(ci, slot)
        if ci >= 2:
            wait_store(ci - 2, slot)
        sv, dv, wv, nv = svs[slot], dvs3[slot], wvs3[slot], nvs[slot]

        @plsc.parallel_loop(jnp.int32(0), jnp.int32(_PREP_NRM_CHUNK // LANES),
                            jnp.int32(1), unroll=UNROLL)
        def vbody3(j):
            sl = pl.ds(j * LANES, LANES)
            g1 = plsc.load_gather(acc_col, [sv[sl]])
            g2 = plsc.load_gather(acc_col, [dv[sl]])
            nv[sl] = g1 * wv[sl] * g2

        o = nbase + ci * _PREP_NRM_CHUNK
        pltpu.async_copy(nvs[slot], out.at[pl.ds(o, _PREP_NRM_CHUNK)],
                         sem_st)
        if ci + 2 < _PREP_NRM_NC:
            issue3(ci + 2, slot)

    for ci in range(max(_PREP_NRM_NC - 2, 0), _PREP_NRM_NC):
        wait_store(ci, ci % 2)


def _proj1_body(x_ref, w_ref, out_ref):
    out_ref[...] = lax.dot_general(
        w_ref[...], x_ref[...], (((0,), (1,)), ((), ())),
        preferred_element_type=jnp.float32)


def _mid_body(y0_ref, t3_ref, sel_ref, b1_ref, w2_ref, out_ref):
    t3 = jnp.where(sel_ref[...] > 0.0, t3_ref[1], t3_ref[0])
    h = y0_ref[...] + t3 + b1_ref[...]
    h = jnp.maximum(h, 0.0)
    out_ref[...] = lax.dot_general(
        w2_ref[...], h, (((1,), (0,)), ((), ())),
        preferred_element_type=jnp.float32)


def _fin_body(v0_ref, q_ref, b2_ref, wl_ref, bl_ref, out_ref):
    o = (v0_ref[...] + jnp.sum(q_ref[...], axis=0, keepdims=True)
         + b2_ref[...])
    o = o * wl_ref[...] + bl_ref[...]
    out_ref[...] = 1.0 / (1.0 + jnp.exp(-o))


_P1 = (4, 12, 0, 16, 8, 4000)
_P2 = (4, 8, 1, 16, 8, 4000)
_P3 = (4, 4, 1, 16, 8, 4000)
_P4 = (1, 3, 0, 4, 32, 2000)
_P5 = (1, 2, 2, 4, 32, 2000)
_P6 = (1, 1, 2, 4, 32, 2000)


def kernel(x, edge_index, edge_attr, W1, b1, W2, b2, Wl, bl):
    src = edge_index[0].astype(jnp.int32)
    dst = edge_index[1].astype(jnp.int32)
    ew = edge_attr.astype(jnp.float32)
    xpad = jnp.zeros((NP, F_IN), jnp.float32).at[:N].set(
        x.astype(jnp.float32))
    wcat1 = jnp.concatenate([W1[0], W1[1], W1[2], W1[3]],
                            axis=1).astype(jnp.float32)

    yt = pl.pallas_call(
        _proj1_body,
        out_shape=jax.ShapeDtypeStruct((16, NP), jnp.float32),
    )(xpad, wcat1)

    nrm = _prep_kernel(src, dst, ew)

    t = _make_prop(*_P1)(yt, src, dst, nrm)
    t = _make_prop(*_P2)(yt, t, src, dst, nrm)
    t = _make_prop(*_P3)(yt, t, src, dst, nrm)

    w2mat = W2[:, :, 0].astype(jnp.float32)
    sel = (jnp.arange(4, dtype=jnp.float32) % 2).reshape(4, 1)
    v = pl.pallas_call(
        _mid_body,
        out_shape=jax.ShapeDtypeStruct((4, NP), jnp.float32),
    )(yt[0:4], t, sel, b1.reshape(4, 1), w2mat)

    q = _make_prop(*_P4)(v, src, dst, nrm)
    q = _make_prop(*_P5)(v, q, src, dst, nrm)
    q = _make_prop(*_P6)(v, q, src, dst, nrm)

    o = pl.pallas_call(
        _fin_body,
        out_shape=jax.ShapeDtypeStruct((1, NP), jnp.float32),
    )(v[0:1], q[:, 0],
      b2.reshape(1, 1).astype(jnp.float32), Wl.astype(jnp.float32),
      bl.reshape(1, 1).astype(jnp.float32))

    return o[0, :N].reshape(N, 1).astype(jnp.float64)


# speedup vs baseline: 759.9808x; 1.0013x over previous
import functools

import jax
import jax.numpy as jnp
from jax import lax
from jax.experimental import pallas as pl
from jax.experimental.pallas import tpu as pltpu
from jax.experimental.pallas import tpu_sc as plsc

N = 10000
NP = 10240
E = 320000
F_IN = 128
LANES = 16
UNROLL = 10

_MESH = plsc.VectorSubcoreMesh(core_axis_name="c", subcore_axis_name="s")
_SC_PARAMS = pltpu.CompilerParams(needs_layout_passes=False)
_NTILES = 32


def _fori(n, body):
    lax.fori_loop(jnp.int32(0), jnp.int32(n), body, jnp.int32(0))


def _zero_col(col):
    z = jnp.zeros((LANES,), jnp.float32)

    def body(i, _):
        base = i * (8 * LANES)
        for u in range(8):
            col[pl.ds(base + u * LANES, LANES)] = z
        return jnp.int32(0)

    _fori(NP // (8 * LANES), body)


def _add_into(dst_col, src_col):
    def body(i, _):
        base = i * (8 * LANES)
        for u in range(8):
            s = pl.ds(base + u * LANES, LANES)
            dst_col[s] = dst_col[s] + src_col[s]
        return jnp.int32(0)

    _fori(NP // (8 * LANES), body)


def _load_col_reduced(base, bcol, part, plist, in_col, tmps, sems):
    pltpu.async_copy(base.at[bcol], in_col, sems[0])
    pltpu.make_async_copy(base.at[bcol], in_col, sems[0]).wait()
    for i, (pi, pc) in enumerate(plist):
        if i == 0:
            pltpu.async_copy(part.at[pi, pc], tmps[0], sems[0])
        if i + 1 < len(plist):
            pi2, pc2 = plist[i + 1]
            s2 = (i + 1) % 2
            pltpu.async_copy(part.at[pi2, pc2], tmps[s2], sems[s2])
        sl = i % 2
        pltpu.make_async_copy(part.at[pi, pc], tmps[sl], sems[sl]).wait()
        _add_into(in_col, tmps[sl])


def _make_prop(d, boff, rp, base_d, nranges, chunk):
    assert d in (1, 4)
    ncpc = max(d // 2, 1)
    nranges = 8 if d == 4 else 32
    er = E // nranges
    nchunks = er // chunk
    gpc = chunk // LANES
    unroll = UNROLL if gpc % UNROLL == 0 else 5
    nper = 16 // ncpc
    segw = NP // nper
    assert er % chunk == 0 and gpc % unroll == 0 and NP % nper == 0

    @functools.partial(
        pl.kernel,
        out_type=jax.ShapeDtypeStruct((2, d, NP), jnp.float32),
        mesh=_MESH,
        compiler_params=_SC_PARAMS,
        scratch_types=(
            [pltpu.VMEM((NP,), jnp.float32) for _ in range(4)]
            + [pltpu.VMEM_SHARED((16, NP), jnp.float32),
               pltpu.VMEM((segw,), jnp.float32),
               pltpu.VMEM((segw,), jnp.float32),
               pltpu.VMEM((chunk,), jnp.int32),
               pltpu.VMEM((chunk,), jnp.int32),
               pltpu.VMEM((chunk,), jnp.int32),
               pltpu.VMEM((chunk,), jnp.int32),
               pltpu.VMEM((chunk,), jnp.float32),
               pltpu.VMEM((chunk,), jnp.float32),
               pltpu.SemaphoreType.DMA,
               pltpu.SemaphoreType.DMA]
        ),
    )
    def prop(*args):
        if rp:
            (base, part, src, dst, nrm, out, in_col, acc_col, tmp0, tmp1,
             shared, seg_a, seg_b,
             src_v0, src_v1, dst_v0, dst_v1, nrm_v0, nrm_v1,
             sem0, sem1) = args
        else:
            (base, src, dst, nrm, out, in_col, acc_col, tmp0, tmp1,
             shared, seg_a, seg_b,
             src_v0, src_v1, dst_v0, dst_v1, nrm_v0, nrm_v1,
             sem0, sem1) = args
            part = None
        tmps = (tmp0, tmp1)
        svs, dvs, nvs = (src_v0, src_v1), (dst_v0, dst_v1), (nrm_v0, nrm_v1)
        sems = (sem0, sem1)
        cid = lax.axis_index("c")
        sid = lax.axis_index("s")

        if True:
            if d == 4:
                cidx = 2 * (sid % ncpc) + cid
                ridx = sid // ncpc
            else:
                cidx = jnp.int32(0)
                ridx = cid * 16 + sid
            if rp == 1:
                plist = [(cidx & 1, cidx)]
            elif rp == 2:
                plist = [(jnp.int32(0), jnp.int32(0)),
                         (jnp.int32(1), jnp.int32(0))]
            else:
                plist = []
            _load_col_reduced(base, jnp.int32(boff) + cidx, part, plist,
                              in_col, tmps, sems)
            _zero_col(acc_col)
            ebase = ridx * er

            def issue(ci, slot):
                o = ebase + ci * chunk
                pltpu.async_copy(src.at[pl.ds(o, chunk)], svs[slot],
                                 sems[slot])
                pltpu.async_copy(dst.at[pl.ds(o, chunk)], dvs[slot],
                                 sems[slot])
                pltpu.async_copy(nrm.at[pl.ds(o, chunk)], nvs[slot],
                                 sems[slot])

            def drain(ci, slot):
                o = ebase + ci * chunk
                pltpu.make_async_copy(src.at[pl.ds(o, chunk)],
                                      svs[slot], sems[slot]).wait()
                pltpu.make_async_copy(dst.at[pl.ds(o, chunk)],
                                      dvs[slot], sems[slot]).wait()
                pltpu.make_async_copy(nrm.at[pl.ds(o, chunk)],
                                      nvs[slot], sems[slot]).wait()

            issue(0, 0)
            if nchunks > 1:
                issue(1, 1)
            for ci in range(nchunks):
                slot = ci % 2
                drain(ci, slot)
                sv, dv, nv = svs[slot], dvs[slot], nvs[slot]

                @plsc.parallel_loop(jnp.int32(0), jnp.int32(gpc),
                                    jnp.int32(1), unroll=unroll)
                def vbody(j):
                    s = pl.ds(j * LANES, LANES)
                    si = sv[s]
                    di = dv[s]
                    w = nv[s]
                    gth = plsc.load_gather(in_col, [si])
                    plsc.addupdate_scatter(acc_col, [di], gth * w)
                if ci + 2 < nchunks:
                    issue(ci + 2, slot)

            pltpu.sync_copy(acc_col, shared.at[sid])
            plsc.subcore_barrier()

            cl2 = sid % ncpc
            k = sid // ncpc
            soff = k * segw
            row0 = cl2
            pltpu.sync_copy(shared.at[row0, pl.ds(soff, segw)], seg_a)
            for p in range(1, nper):
                row = cl2 + ncpc * p
                pltpu.sync_copy(shared.at[row, pl.ds(soff, segw)], seg_b)

                def sbody(i, _):
                    b2 = i * (8 * LANES)
                    for u in range(8):
                        sl = pl.ds(b2 + u * LANES, LANES)
                        seg_a[sl] = seg_a[sl] + seg_b[sl]
                    return jnp.int32(0)

                _fori(segw // (8 * LANES), sbody)

            ocol = 2 * cl2 + cid if d == 4 else jnp.int32(0)
            pltpu.sync_copy(seg_a, out.at[cid, ocol, pl.ds(soff, segw)])

    return prop


_PREP_DEG_ER = E // 16
_PREP_DEG_CHUNK = 4000
_PREP_DEG_NC = _PREP_DEG_ER // _PREP_DEG_CHUNK
_PREP_NRM_ER = E // 32
_PREP_NRM_CHUNK = 2000
_PREP_NRM_NC = _PREP_NRM_ER // _PREP_NRM_CHUNK
_SEGW = NP // 16


@functools.partial(
    pl.kernel,
    out_type=jax.ShapeDtypeStruct((E,), jnp.float32),
    mesh=_MESH,
    compiler_params=_SC_PARAMS,
    scratch_types=[
        pltpu.VMEM((NP,), jnp.float32),
        pltpu.VMEM_SHARED((16, NP), jnp.float32),
        pltpu.VMEM_SHARED((NP,), jnp.float32),
        pltpu.VMEM((_SEGW,), jnp.float32),
        pltpu.VMEM((_SEGW,), jnp.float32),
        pltpu.VMEM((_PREP_DEG_CHUNK,), jnp.int32),
        pltpu.VMEM((_PREP_DEG_CHUNK,), jnp.int32),
        pltpu.VMEM((_PREP_DEG_CHUNK,), jnp.float32),
        pltpu.VMEM((_PREP_DEG_CHUNK,), jnp.float32),
        pltpu.VMEM((_PREP_NRM_CHUNK,), jnp.int32),
        pltpu.VMEM((_PREP_NRM_CHUNK,), jnp.int32),
        pltpu.VMEM((_PREP_NRM_CHUNK,), jnp.int32),
        pltpu.VMEM((_PREP_NRM_CHUNK,), jnp.int32),
        pltpu.VMEM((_PREP_NRM_CHUNK,), jnp.float32),
        pltpu.VMEM((_PREP_NRM_CHUNK,), jnp.float32),
        pltpu.VMEM((_PREP_NRM_CHUNK,), jnp.float32),
        pltpu.VMEM((_PREP_NRM_CHUNK,), jnp.float32),
        pltpu.SemaphoreType.DMA,
        pltpu.SemaphoreType.DMA,
        pltpu.SemaphoreType.DMA,
    ],
)
def _prep_kernel(src, dst, ew, out, acc_col, shared, shared_dis, seg_a,
                 seg_b, d1_v0, d1_v1, w1_v0, w1_v1, s3_v0, s3_v1,
                 d3_v0, d3_v1, w3_v0, w3_v1, n3_v0, n3_v1,
                 sem0, sem1, sem_st):
    cid = lax.axis_index("c")
    sid = lax.axis_index("s")
    sems = (sem0, sem1)

    _zero_col(acc_col)
    dvs, wvs = (d1_v0, d1_v1), (w1_v0, w1_v1)
    ebase = sid * _PREP_DEG_ER

    def issue1(ci, slot):
        o = ebase + ci * _PREP_DEG_CHUNK
        pltpu.async_copy(dst.at[pl.ds(o, _PREP_DEG_CHUNK)], dvs[slot],
                         sems[slot])
        pltpu.async_copy(ew.at[pl.ds(o, _PREP_DEG_CHUNK)], wvs[slot],
                         sems[slot])

    def drain1(ci, slot):
        o = ebase + ci * _PREP_DEG_CHUNK
        pltpu.make_async_copy(dst.at[pl.ds(o, _PREP_DEG_CHUNK)], dvs[slot],
                              sems[slot]).wait()
        pltpu.make_async_copy(ew.at[pl.ds(o, _PREP_DEG_CHUNK)], wvs[slot],
                              sems[slot]).wait()

    issue1(0, 0)
    issue1(1, 1)
    for ci in range(_PREP_DEG_NC):
        slot = ci % 2
        drain1(ci, slot)
        dv, wv = dvs[slot], wvs[slot]

        @plsc.parallel_loop(jnp.int32(0), jnp.int32(_PREP_DEG_CHUNK // LANES),
                            jnp.int32(1), unroll=UNROLL)
        def vbody(j):
            sl = pl.ds(j * LANES, LANES)
            plsc.addupdate_scatter(acc_col, [dv[sl]], wv[sl])

        if ci + 2 < _PREP_DEG_NC:
            issue1(ci + 2, slot)

    pltpu.sync_copy(acc_col, shared.at[sid])
    plsc.subcore_barrier()

    soff = sid * _SEGW
    pltpu.sync_copy(shared.at[jnp.int32(0), pl.ds(soff, _SEGW)], seg_a)
    for p in range(1, 16):
        pltpu.sync_copy(shared.at[jnp.int32(p), pl.ds(soff, _SEGW)], seg_b)

        def sbody(i, _):
            b2 = i * (8 * LANES)
            for u in range(8):
                sl = pl.ds(b2 + u * LANES, LANES)
                seg_a[sl] = seg_a[sl] + seg_b[sl]
            return jnp.int32(0)

        _fori(_SEGW // (8 * LANES), sbody)

    def rbody(i, _):
        sl = pl.ds(i * LANES, LANES)
        d = seg_a[sl]
        bits = plsc.bitcast(d, jnp.int32)
        y = plsc.bitcast(jnp.int32(0x5F3759DF) -
                         lax.shift_right_logical(bits, jnp.int32(1)),
                         jnp.float32)
        hd = d * 0.5
        for _unused in range(3):
            y = y * (1.5 - hd * y * y)
        seg_a[sl] = jnp.where(d > 0.0, y, 0.0)
        return jnp.int32(0)

    _fori(_SEGW // LANES, rbody)
    pltpu.sync_copy(seg_a, shared_dis.at[pl.ds(soff, _SEGW)])
    plsc.subcore_barrier()

    pltpu.sync_copy(shared_dis, acc_col)
    svs, dvs3 = (s3_v0, s3_v1), (d3_v0, d3_v1)
    wvs3, nvs = (w3_v0, w3_v1), (n3_v0, n3_v1)
    nbase = (cid * 16 + sid) * _PREP_NRM_ER

    def issue3(ci, slot):
        o = nbase + ci * _PREP_NRM_CHUNK
        pltpu.async_copy(src.at[pl.ds(o, _PREP_NRM_CHUNK)], svs[slot],
                         sems[slot])
        pltpu.async_copy(dst.at[pl.ds(o, _PREP_NRM_CHUNK)], dvs3[slot],
                         sems[slot])
        pltpu.async_copy(ew.at[pl.ds(o, _PREP_NRM_CHUNK)], wvs3[slot],
                         sems[slot])

    def drain3(ci, slot):
        o = nbase + ci * _PREP_NRM_CHUNK
        pltpu.make_async_copy(src.at[pl.ds(o, _PREP_NRM_CHUNK)], svs[slot],
                              sems[slot]).wait()
        pltpu.make_async_copy(dst.at[pl.ds(o, _PREP_NRM_CHUNK)], dvs3[slot],
                              sems[slot]).wait()
        pltpu.make_async_copy(ew.at[pl.ds(o, _PREP_NRM_CHUNK)], wvs3[slot],
                              sems[slot]).wait()

    def wait_store(ci, slot):
        o = nbase + ci * _PREP_NRM_CHUNK
        pltpu.make_async_copy(nvs[slot], out.at[pl.ds(o, _PREP_NRM_CHUNK)],
                              sem_st).wait()

    issue3(0, 0)
    issue3(1, 1)
    for ci in range(_PREP_NRM_NC):
        slot = ci % 2
        drain3(ci, slot)
        if ci >= 2:
            wait_store(ci - 2, slot)
        sv, dv, wv, nv = svs[slot], dvs3[slot], wvs3[slot], nvs[slot]

        @plsc.parallel_loop(jnp.int32(0), jnp.int32(_PREP_NRM_CHUNK // LANES),
                            jnp.int32(1), unroll=UNROLL)
        def vbody3(j):
            sl = pl.ds(j * LANES, LANES)
            g1 = plsc.load_gather(acc_col, [sv[sl]])
            g2 = plsc.load_gather(acc_col, [dv[sl]])
            nv[sl] = g1 * wv[sl] * g2

        o = nbase + ci * _PREP_NRM_CHUNK
        pltpu.async_copy(nvs[slot], out.at[pl.ds(o, _PREP_NRM_CHUNK)],
                         sem_st)
        if ci + 2 < _PREP_NRM_NC:
            issue3(ci + 2, slot)

    for ci in range(max(_PREP_NRM_NC - 2, 0), _PREP_NRM_NC):
        wait_store(ci, ci % 2)


def _proj1_body(x_ref, w_ref, out_ref):
    out_ref[...] = lax.dot_general(
        w_ref[...], x_ref[...], (((0,), (1,)), ((), ())),
        preferred_element_type=jnp.float32)


def _mid_body(y0_ref, t3_ref, sel_ref, b1_ref, w2_ref, out_ref):
    t3 = jnp.where(sel_ref[...] > 0.0, t3_ref[1], t3_ref[0])
    h = y0_ref[...] + t3 + b1_ref[...]
    h = jnp.maximum(h, 0.0)
    out_ref[...] = lax.dot_general(
        w2_ref[...], h, (((1,), (0,)), ((), ())),
        preferred_element_type=jnp.float32)


def _fin_body(v0_ref, q_ref, b2_ref, wl_ref, bl_ref, out_ref):
    o = (v0_ref[...] + jnp.sum(q_ref[...], axis=0, keepdims=True)
         + b2_ref[...])
    o = o * wl_ref[...] + bl_ref[...]
    out_ref[...] = 1.0 / (1.0 + jnp.exp(-o))


_P1 = (4, 12, 0, 16, 8, 4000)
_P2 = (4, 8, 1, 16, 8, 4000)
_P3 = (4, 4, 1, 16, 8, 4000)
_P4 = (1, 3, 0, 4, 32, 2000)
_P5 = (1, 2, 2, 4, 32, 2000)
_P6 = (1, 1, 2, 4, 32, 2000)


def kernel(x, edge_index, edge_attr, W1, b1, W2, b2, Wl, bl):
    src = edge_index[0].astype(jnp.int32)
    dst = edge_index[1].astype(jnp.int32)
    ew = edge_attr.astype(jnp.float32)
    xpad = jnp.zeros((NP, F_IN), jnp.float32).at[:N].set(
        x.astype(jnp.float32))
    wcat1 = jnp.concatenate([W1[0], W1[1], W1[2], W1[3]],
                            axis=1).astype(jnp.float32)

    yt = pl.pallas_call(
        _proj1_body,
        out_shape=jax.ShapeDtypeStruct((16, NP), jnp.float32),
    )(xpad, wcat1)

    nrm = _prep_kernel(src, dst, ew)

    t = _make_prop(*_P1)(yt, src, dst, nrm)
    t = _make_prop(*_P2)(yt, t, src, dst, nrm)
    t = _make_prop(*_P3)(yt, t, src, dst, nrm)

    w2mat = W2[:, :, 0].astype(jnp.float32)
    sel = (jnp.arange(4, dtype=jnp.float32) % 2).reshape(4, 1)
    v = pl.pallas_call(
        _mid_body,
        out_shape=jax.ShapeDtypeStruct((4, NP), jnp.float32),
    )(yt[0:4], t, sel, b1.reshape(4, 1), w2mat)

    q = _make_prop(*_P4)(v, src, dst, nrm)
    q = _make_prop(*_P5)(v, q, src, dst, nrm)
    q = _make_prop(*_P6)(v, q, src, dst, nrm)

    o = pl.pallas_call(
        _fin_body,
        out_shape=jax.ShapeDtypeStruct((1, NP), jnp.float32),
    )(v[0:1], q[:, 0],
      b2.reshape(1, 1).astype(jnp.float32), Wl.astype(jnp.float32),
      bl.reshape(1, 1).astype(jnp.float32))

    return o[0, :N].reshape(N, 1).astype(jnp.float64)
